# baseline (speedup 1.0000x reference)
import jax
import jax.numpy as jnp
from jax.experimental import pallas as pl
from jax.experimental.pallas import tpu as pltpu

N_NODES = 10000
N_EDGES = 4096
D = 128
H = 8
DH = 16
NP = 10240


def _ln_rows(xv, g, b):
    m = jnp.mean(xv, axis=-1, keepdims=True)
    v = jnp.mean((xv - m) ** 2, axis=-1, keepdims=True)
    return (xv - m) * jax.lax.rsqrt(v + 1e-5) * g + b


def _src_prep(h, Wv_b, P128, ns, bt):
    npad = h.shape[0]
    grid = (npad // bt,)

    def body(h_ref, wv_ref, p_ref, l_ref, v_ref, mx_ref):
        i = pl.program_id(0)
        h_ = h_ref[...]
        hv = jnp.dot(h_, wv_ref[...], preferred_element_type=jnp.float32)
        l = jnp.dot(h_, p_ref[...], preferred_element_type=jnp.float32)
        row = jax.lax.broadcasted_iota(jnp.int32, (bt, D), 0)
        valid = (i * bt + row) < ns
        l = jnp.where(valid, l, 0.0)
        hv = jnp.where(valid, hv, 0.0)
        l_ref[...] = l
        v_ref[...] = hv
        tmax = jnp.max(jnp.where(valid, l, -1e30), axis=0, keepdims=True)
        tmax = jnp.broadcast_to(tmax, (8, D))

        @pl.when(i == 0)
        def _():
            mx_ref[...] = jnp.full((8, D), -1e30, jnp.float32)

        mx_ref[...] = jnp.maximum(mx_ref[...], tmax)

    return pl.pallas_call(
        body,
        grid=grid,
        in_specs=[
            pl.BlockSpec((bt, D), lambda i: (i, 0)),
            pl.BlockSpec((D, D), lambda i: (0, 0)),
            pl.BlockSpec((D, D), lambda i: (0, 0)),
        ],
        out_specs=[
            pl.BlockSpec((bt, D), lambda i: (i, 0)),
            pl.BlockSpec((bt, D), lambda i: (i, 0)),
            pl.BlockSpec((8, D), lambda i: (0, 0)),
        ],
        out_shape=[
            jax.ShapeDtypeStruct((npad, D), jnp.float32),
            jax.ShapeDtypeStruct((npad, D), jnp.float32),
            jax.ShapeDtypeStruct((8, D), jnp.float32),
        ],
    )(h, Wv_b, P128)


def _pool_e_from_n(inc, l128, hv, mx, W1_b, W2_b, small, bm, bk,
                   emit_bf16=False, classify=False, Wc1=None, Wc2p=None):
    mt = N_EDGES // bm
    kt = NP // bk

    def body(inc_ref, l_ref, v_ref, mx_ref, w1_ref, w2_ref, s_ref, *rest):
        if classify:
            wc1_ref, wc2_ref = rest[:2]
            rest = rest[2:]
        if emit_bf16:
            out_ref, cpy_ref, num, den = rest
        else:
            out_ref, num, den = rest
        k = pl.program_id(1)

        @pl.when(k == 0)
        def _():
            num[...] = jnp.zeros_like(num)
            den[...] = jnp.zeros_like(den)

        if emit_bf16:
            lhs = inc_ref[...]
            col = jax.lax.broadcasted_iota(jnp.int32, (bm, bk), 1)
            lhs = jnp.where(k * bk + col < N_NODES, lhs, 0.0)
            lhs = lhs.astype(jnp.bfloat16)
            cpy_ref[...] = lhs
        else:
            lhs = inc_ref[...]
        mxv = jnp.max(mx_ref[...], axis=0, keepdims=True)
        w = jnp.exp(l_ref[pl.ds(k * bk, bk), :] - mxv)
        wv = (w * v_ref[pl.ds(k * bk, bk), :]).astype(jnp.bfloat16)
        w = w.astype(jnp.bfloat16)
        num[...] += jnp.dot(lhs, wv, preferred_element_type=jnp.float32)
        den[...] += jnp.dot(lhs, w, preferred_element_type=jnp.float32)

        @pl.when(k == kt - 1)
        def _():
            s = s_ref[...]
            pooled = num[...] / (den[...] + 1e-9)
            y = _ln_rows(pooled, s[0:1, :], s[1:2, :])
            y2 = jnp.dot(
                jax.nn.relu(jnp.dot(y, w1_ref[...],
                                    preferred_element_type=jnp.float32)),
                w2_ref[...], preferred_element_type=jnp.float32)
            o = _ln_rows(y + y2, s[2:3, :], s[3:4, :])
            if classify:
                hcl = jax.nn.relu(
                    jnp.dot(o, wc1_ref[...],
                            preferred_element_type=jnp.float32) + s[4:5, :])
                o = jnp.dot(hcl, wc2_ref[...],
                            preferred_element_type=jnp.float32) + s[5:6, :]
            out_ref[...] = o

    in_specs = [
        pl.BlockSpec((bm, bk), lambda m, k: (m, k)),
        pl.BlockSpec((NP, D), lambda m, k: (0, 0)),
        pl.BlockSpec((NP, D), lambda m, k: (0, 0)),
        pl.BlockSpec((8, D), lambda m, k: (0, 0)),
        pl.BlockSpec((D, D), lambda m, k: (0, 0)),
        pl.BlockSpec((D, D), lambda m, k: (0, 0)),
        pl.BlockSpec((8, D), lambda m, k: (0, 0)),
    ]
    args = [inc, l128, hv, mx, W1_b, W2_b, small]
    if classify:
        in_specs += [pl.BlockSpec((D, D), lambda m, k: (0, 0)),
                     pl.BlockSpec((D, D), lambda m, k: (0, 0))]
        args += [Wc1, Wc2p]

    out_specs = [pl.BlockSpec((bm, D), lambda m, k: (m, 0))]
    out_shape = [jax.ShapeDtypeStruct((N_EDGES, D), jnp.float32)]
    if emit_bf16:
        out_specs.append(pl.BlockSpec((bm, bk), lambda m, k: (m, k)))
        out_shape.append(jax.ShapeDtypeStruct((N_EDGES, NP), jnp.bfloat16))

    res = pl.pallas_call(
        body,
        grid=(mt, kt),
        in_specs=in_specs,
        out_specs=out_specs,
        out_shape=out_shape,
        scratch_shapes=[pltpu.VMEM((bm, D), jnp.float32),
                        pltpu.VMEM((bm, D), jnp.float32)],
    )(*args)
    return res if emit_bf16 else res[0]


def _pool_n_from_e(inc_bf16, l128, hv, mx, W1_b, W2_b, small, bm, bk):
    mt = NP // bm
    kt = N_EDGES // bk
    dn = (((0,), (0,)), ((), ()))

    def body(inc_ref, l_ref, v_ref, mx_ref, w1_ref, w2_ref, s_ref,
             out_ref, num, den):
        k = pl.program_id(1)

        @pl.when(k == 0)
        def _():
            num[...] = jnp.zeros_like(num)
            den[...] = jnp.zeros_like(den)

        lhs = inc_ref[...]
        mxv = jnp.max(mx_ref[...], axis=0, keepdims=True)
        w = jnp.exp(l_ref[pl.ds(k * bk, bk), :] - mxv)
        wv = (w * v_ref[pl.ds(k * bk, bk), :]).astype(jnp.bfloat16)
        w = w.astype(jnp.bfloat16)
        num[...] += jax.lax.dot_general(lhs, wv, dn,
                                        preferred_element_type=jnp.float32)
        den[...] += jax.lax.dot_general(lhs, w, dn,
                                        preferred_element_type=jnp.float32)

        @pl.when(k == kt - 1)
        def _():
            s = s_ref[...]
            pooled = num[...] / (den[...] + 1e-9)
            y = _ln_rows(pooled, s[0:1, :], s[1:2, :])
            y2 = jnp.dot(
                jax.nn.relu(jnp.dot(y, w1_ref[...],
                                    preferred_element_type=jnp.float32)),
                w2_ref[...], preferred_element_type=jnp.float32)
            out_ref[...] = _ln_rows(y + y2, s[2:3, :], s[3:4, :])

    return pl.pallas_call(
        body,
        grid=(mt, kt),
        in_specs=[
            pl.BlockSpec((bk, bm), lambda m, k: (k, m)),
            pl.BlockSpec((N_EDGES, D), lambda m, k: (0, 0)),
            pl.BlockSpec((N_EDGES, D), lambda m, k: (0, 0)),
            pl.BlockSpec((8, D), lambda m, k: (0, 0)),
            pl.BlockSpec((D, D), lambda m, k: (0, 0)),
            pl.BlockSpec((D, D), lambda m, k: (0, 0)),
            pl.BlockSpec((8, D), lambda m, k: (0, 0)),
        ],
        out_specs=pl.BlockSpec((bm, D), lambda m, k: (m, 0)),
        out_shape=jax.ShapeDtypeStruct((NP, D), jnp.float32),
        scratch_shapes=[pltpu.VMEM((bm, D), jnp.float32),
                        pltpu.VMEM((bm, D), jnp.float32)],
    )(inc_bf16, l128, hv, mx, W1_b, W2_b, small)


def kernel(x, incidence_matrix, Wk, Wv, q, W1, W2, ln_g, ln_b,
           Wc1, bc1, Wc2, bc2):
    f32 = jnp.float32
    scale = jnp.sqrt(jnp.asarray(DH, f32))
    x_p = jnp.pad(x, ((0, NP - N_NODES), (0, 0)))

    def p128(b):
        qexp = jnp.zeros((H, DH, H), f32)
        qexp = qexp.at[jnp.arange(H), :, jnp.arange(H)].set(q[b])
        qexp = qexp.reshape(D, H)
        P = (Wk[b] @ qexp) / scale
        return jnp.repeat(P, DH, axis=1)

    zero = jnp.zeros((D,), f32)

    def small(b, classify=False):
        rows = [ln_g[b, 0], ln_b[b, 0], ln_g[b, 1], ln_b[b, 1]]
        if classify:
            rows += [bc1, jnp.broadcast_to(bc2, (D,))]
        else:
            rows += [zero, zero]
        rows += [zero, zero]
        return jnp.stack(rows)

    l0, v0, m0 = _src_prep(x_p, Wv[0], p128(0), N_NODES, 2048)
    h1, inc_bf16 = _pool_e_from_n(incidence_matrix, l0, v0, m0,
                                  W1[0], W2[0], small(0),
                                  bm=1024, bk=2048, emit_bf16=True)
    l1, v1, m1 = _src_prep(h1, Wv[1], p128(1), N_EDGES, 2048)
    h0 = _pool_n_from_e(inc_bf16, l1, v1, m1, W1[1], W2[1],
                        small(1), bm=2048, bk=1024)
    l2, v2, m2 = _src_prep(h0, Wv[2], p128(2), N_NODES, 2048)
    Wc2p = jnp.pad(Wc2, ((0, 0), (0, D - 1)))
    res = _pool_e_from_n(inc_bf16, l2, v2, m2, W1[2], W2[2],
                         small(2, classify=True), bm=1024, bk=2048,
                         classify=True, Wc1=Wc1, Wc2p=Wc2p)
    return res[:, :1]


# speedup vs baseline: 1.0014x; 1.0014x over previous
import jax
import jax.numpy as jnp
from jax.experimental import pallas as pl
from jax.experimental.pallas import tpu as pltpu

N_NODES = 10000
N_EDGES = 4096
D = 128
H = 8
DH = 16
NP = 10240


def _ln_rows(xv, g, b):
    m = jnp.mean(xv, axis=-1, keepdims=True)
    v = jnp.mean((xv - m) ** 2, axis=-1, keepdims=True)
    return (xv - m) * jax.lax.rsqrt(v + 1e-5) * g + b


def _src_prep(h, Wv_b, P128, ns, bt):
    npad = h.shape[0]
    grid = (npad // bt,)

    def body(h_ref, wv_ref, p_ref, l_ref, v_ref, mx_ref):
        i = pl.program_id(0)
        h_ = h_ref[...]
        hv = jnp.dot(h_, wv_ref[...], preferred_element_type=jnp.float32)
        l = jnp.dot(h_, p_ref[...], preferred_element_type=jnp.float32)
        row = jax.lax.broadcasted_iota(jnp.int32, (bt, D), 0)
        valid = (i * bt + row) < ns
        l = jnp.where(valid, l, 0.0)
        hv = jnp.where(valid, hv, 0.0)
        l_ref[...] = l
        v_ref[...] = hv
        tmax = jnp.max(jnp.where(valid, l, -1e30), axis=0, keepdims=True)
        tmax = jnp.broadcast_to(tmax, (8, D))

        @pl.when(i == 0)
        def _():
            mx_ref[...] = jnp.full((8, D), -1e30, jnp.float32)

        mx_ref[...] = jnp.maximum(mx_ref[...], tmax)

    return pl.pallas_call(
        body,
        grid=grid,
        in_specs=[
            pl.BlockSpec((bt, D), lambda i: (i, 0)),
            pl.BlockSpec((D, D), lambda i: (0, 0)),
            pl.BlockSpec((D, D), lambda i: (0, 0)),
        ],
        out_specs=[
            pl.BlockSpec((bt, D), lambda i: (i, 0)),
            pl.BlockSpec((bt, D), lambda i: (i, 0)),
            pl.BlockSpec((8, D), lambda i: (0, 0)),
        ],
        out_shape=[
            jax.ShapeDtypeStruct((npad, D), jnp.float32),
            jax.ShapeDtypeStruct((npad, D), jnp.float32),
            jax.ShapeDtypeStruct((8, D), jnp.float32),
        ],
    )(h, Wv_b, P128)


def _pool_e_from_n(inc, l128, hv, mx, W1_b, W2_b, small, bm, bk,
                   emit_bf16=False, classify=False, Wc1=None, Wc2p=None):
    mt = N_EDGES // bm
    kt = NP // bk

    def body(inc_ref, l_ref, v_ref, mx_ref, w1_ref, w2_ref, s_ref, *rest):
        if classify:
            wc1_ref, wc2_ref = rest[:2]
            rest = rest[2:]
        if emit_bf16:
            out_ref, cpy_ref, num, den = rest
        else:
            out_ref, num, den = rest
        k = pl.program_id(1)

        @pl.when(k == 0)
        def _():
            num[...] = jnp.zeros_like(num)
            den[...] = jnp.zeros_like(den)

        if emit_bf16:
            lhs = inc_ref[...]
            col = jax.lax.broadcasted_iota(jnp.int32, (bm, bk), 1)
            lhs = jnp.where(k * bk + col < N_NODES, lhs, 0.0)
            lhs = lhs.astype(jnp.bfloat16)
            cpy_ref[...] = lhs
        else:
            lhs = inc_ref[...]
        mxv = jnp.max(mx_ref[...], axis=0, keepdims=True)
        w = jnp.exp(l_ref[pl.ds(k * bk, bk), :] - mxv)
        wv = (w * v_ref[pl.ds(k * bk, bk), :]).astype(jnp.bfloat16)
        w = w.astype(jnp.bfloat16)
        num[...] += jnp.dot(lhs, wv, preferred_element_type=jnp.float32)
        den[...] += jnp.dot(lhs, w, preferred_element_type=jnp.float32)

        @pl.when(k == kt - 1)
        def _():
            s = s_ref[...]
            pooled = num[...] / (den[...] + 1e-9)
            y = _ln_rows(pooled, s[0:1, :], s[1:2, :])
            y2 = jnp.dot(
                jax.nn.relu(jnp.dot(y, w1_ref[...],
                                    preferred_element_type=jnp.float32)),
                w2_ref[...], preferred_element_type=jnp.float32)
            o = _ln_rows(y + y2, s[2:3, :], s[3:4, :])
            if classify:
                hcl = jax.nn.relu(
                    jnp.dot(o, wc1_ref[...],
                            preferred_element_type=jnp.float32) + s[4:5, :])
                o = jnp.dot(hcl, wc2_ref[...],
                            preferred_element_type=jnp.float32) + s[5:6, :]
            out_ref[...] = o

    in_specs = [
        pl.BlockSpec((bm, bk), lambda m, k: (m, k)),
        pl.BlockSpec((NP, D), lambda m, k: (0, 0)),
        pl.BlockSpec((NP, D), lambda m, k: (0, 0)),
        pl.BlockSpec((8, D), lambda m, k: (0, 0)),
        pl.BlockSpec((D, D), lambda m, k: (0, 0)),
        pl.BlockSpec((D, D), lambda m, k: (0, 0)),
        pl.BlockSpec((8, D), lambda m, k: (0, 0)),
    ]
    args = [inc, l128, hv, mx, W1_b, W2_b, small]
    if classify:
        in_specs += [pl.BlockSpec((D, D), lambda m, k: (0, 0)),
                     pl.BlockSpec((D, D), lambda m, k: (0, 0))]
        args += [Wc1, Wc2p]

    out_specs = [pl.BlockSpec((bm, D), lambda m, k: (m, 0))]
    out_shape = [jax.ShapeDtypeStruct((N_EDGES, D), jnp.float32)]
    if emit_bf16:
        out_specs.append(pl.BlockSpec((bm, bk), lambda m, k: (m, k)))
        out_shape.append(jax.ShapeDtypeStruct((N_EDGES, NP), jnp.bfloat16))

    res = pl.pallas_call(
        body,
        grid=(mt, kt),
        in_specs=in_specs,
        out_specs=out_specs,
        out_shape=out_shape,
        scratch_shapes=[pltpu.VMEM((bm, D), jnp.float32),
                        pltpu.VMEM((bm, D), jnp.float32)],
        compiler_params=pltpu.CompilerParams(
            dimension_semantics=("parallel", "arbitrary")),
    )(*args)
    return res if emit_bf16 else res[0]


def _pool_n_from_e(inc_bf16, l128, hv, mx, W1_b, W2_b, small, bm, bk):
    mt = NP // bm
    kt = N_EDGES // bk
    dn = (((0,), (0,)), ((), ()))

    def body(inc_ref, l_ref, v_ref, mx_ref, w1_ref, w2_ref, s_ref,
             out_ref, num, den):
        k = pl.program_id(1)

        @pl.when(k == 0)
        def _():
            num[...] = jnp.zeros_like(num)
            den[...] = jnp.zeros_like(den)

        lhs = inc_ref[...]
        mxv = jnp.max(mx_ref[...], axis=0, keepdims=True)
        w = jnp.exp(l_ref[pl.ds(k * bk, bk), :] - mxv)
        wv = (w * v_ref[pl.ds(k * bk, bk), :]).astype(jnp.bfloat16)
        w = w.astype(jnp.bfloat16)
        num[...] += jax.lax.dot_general(lhs, wv, dn,
                                        preferred_element_type=jnp.float32)
        den[...] += jax.lax.dot_general(lhs, w, dn,
                                        preferred_element_type=jnp.float32)

        @pl.when(k == kt - 1)
        def _():
            s = s_ref[...]
            pooled = num[...] / (den[...] + 1e-9)
            y = _ln_rows(pooled, s[0:1, :], s[1:2, :])
            y2 = jnp.dot(
                jax.nn.relu(jnp.dot(y, w1_ref[...],
                                    preferred_element_type=jnp.float32)),
                w2_ref[...], preferred_element_type=jnp.float32)
            out_ref[...] = _ln_rows(y + y2, s[2:3, :], s[3:4, :])

    return pl.pallas_call(
        body,
        grid=(mt, kt),
        in_specs=[
            pl.BlockSpec((bk, bm), lambda m, k: (k, m)),
            pl.BlockSpec((N_EDGES, D), lambda m, k: (0, 0)),
            pl.BlockSpec((N_EDGES, D), lambda m, k: (0, 0)),
            pl.BlockSpec((8, D), lambda m, k: (0, 0)),
            pl.BlockSpec((D, D), lambda m, k: (0, 0)),
            pl.BlockSpec((D, D), lambda m, k: (0, 0)),
            pl.BlockSpec((8, D), lambda m, k: (0, 0)),
        ],
        out_specs=pl.BlockSpec((bm, D), lambda m, k: (m, 0)),
        out_shape=jax.ShapeDtypeStruct((NP, D), jnp.float32),
        scratch_shapes=[pltpu.VMEM((bm, D), jnp.float32),
                        pltpu.VMEM((bm, D), jnp.float32)],
        compiler_params=pltpu.CompilerParams(
            dimension_semantics=("parallel", "arbitrary")),
    )(inc_bf16, l128, hv, mx, W1_b, W2_b, small)


def kernel(x, incidence_matrix, Wk, Wv, q, W1, W2, ln_g, ln_b,
           Wc1, bc1, Wc2, bc2):
    f32 = jnp.float32
    scale = jnp.sqrt(jnp.asarray(DH, f32))
    x_p = jnp.pad(x, ((0, NP - N_NODES), (0, 0)))

    def p128(b):
        qexp = jnp.zeros((H, DH, H), f32)
        qexp = qexp.at[jnp.arange(H), :, jnp.arange(H)].set(q[b])
        qexp = qexp.reshape(D, H)
        P = (Wk[b] @ qexp) / scale
        return jnp.repeat(P, DH, axis=1)

    zero = jnp.zeros((D,), f32)

    def small(b, classify=False):
        rows = [ln_g[b, 0], ln_b[b, 0], ln_g[b, 1], ln_b[b, 1]]
        if classify:
            rows += [bc1, jnp.broadcast_to(bc2, (D,))]
        else:
            rows += [zero, zero]
        rows += [zero, zero]
        return jnp.stack(rows)

    l0, v0, m0 = _src_prep(x_p, Wv[0], p128(0), N_NODES, 2048)
    h1, inc_bf16 = _pool_e_from_n(incidence_matrix, l0, v0, m0,
                                  W1[0], W2[0], small(0),
                                  bm=1024, bk=2048, emit_bf16=True)
    l1, v1, m1 = _src_prep(h1, Wv[1], p128(1), N_EDGES, 2048)
    h0 = _pool_n_from_e(inc_bf16, l1, v1, m1, W1[1], W2[1],
                        small(1), bm=2048, bk=1024)
    l2, v2, m2 = _src_prep(h0, Wv[2], p128(2), N_NODES, 2048)
    Wc2p = jnp.pad(Wc2, ((0, 0), (0, D - 1)))
    res = _pool_e_from_n(inc_bf16, l2, v2, m2, W1[2], W2[2],
                         small(2, classify=True), bm=1024, bk=2048,
                         classify=True, Wc1=Wc1, Wc2p=Wc2p)
    return res[:, :1]


# speedup vs baseline: 1.5624x; 1.5602x over previous
import jax
import jax.numpy as jnp
from jax.experimental import pallas as pl
from jax.experimental.pallas import tpu as pltpu

N_NODES = 10000
N_EDGES = 4096
D = 128
H = 8
DH = 16
NP = 10240


def _ln_rows(xv, g, b):
    m = jnp.mean(xv, axis=-1, keepdims=True)
    v = jnp.mean((xv - m) ** 2, axis=-1, keepdims=True)
    return (xv - m) * jax.lax.rsqrt(v + 1e-5) * g + b


def _src_prep(h, Wv_b, P128, ns, bt):
    npad = h.shape[0]
    grid = (npad // bt,)

    def body(h_ref, wv_ref, p_ref, l_ref, v_ref, mx_ref):
        i = pl.program_id(0)
        h_ = h_ref[...]
        hv = jnp.dot(h_, wv_ref[...], preferred_element_type=jnp.float32)
        l = jnp.dot(h_, p_ref[...], preferred_element_type=jnp.float32)
        row = jax.lax.broadcasted_iota(jnp.int32, (bt, D), 0)
        valid = (i * bt + row) < ns
        l = jnp.where(valid, l, 0.0)
        hv = jnp.where(valid, hv, 0.0)
        l_ref[...] = l
        v_ref[...] = hv
        tmax = jnp.max(jnp.where(valid, l, -1e30), axis=0, keepdims=True)
        tmax = jnp.broadcast_to(tmax, (8, D))

        @pl.when(i == 0)
        def _():
            mx_ref[...] = jnp.full((8, D), -1e30, jnp.float32)

        mx_ref[...] = jnp.maximum(mx_ref[...], tmax)

    return pl.pallas_call(
        body,
        grid=grid,
        in_specs=[
            pl.BlockSpec((bt, D), lambda i: (i, 0)),
            pl.BlockSpec((D, D), lambda i: (0, 0)),
            pl.BlockSpec((D, D), lambda i: (0, 0)),
        ],
        out_specs=[
            pl.BlockSpec((bt, D), lambda i: (i, 0)),
            pl.BlockSpec((bt, D), lambda i: (i, 0)),
            pl.BlockSpec((8, D), lambda i: (0, 0)),
        ],
        out_shape=[
            jax.ShapeDtypeStruct((npad, D), jnp.float32),
            jax.ShapeDtypeStruct((npad, D), jnp.float32),
            jax.ShapeDtypeStruct((8, D), jnp.float32),
        ],
    )(h, Wv_b, P128)


def _pool_e_from_n(incT, l128, hv, mx, W1_b, W2_b, small, bm, bk,
                   emit_bf16=False, classify=False, Wc1=None, Wc2p=None):
    mt = N_EDGES // bm
    kt = NP // bk
    dn = (((0,), (0,)), ((), ()))

    def body(inc_ref, l_ref, v_ref, mx_ref, w1_ref, w2_ref, s_ref, *rest):
        if classify:
            wc1_ref, wc2_ref = rest[:2]
            rest = rest[2:]
        if emit_bf16:
            out_ref, cpy_ref, num, den = rest
        else:
            out_ref, num, den = rest
        k = pl.program_id(1)

        @pl.when(k == 0)
        def _():
            num[...] = jnp.zeros_like(num)
            den[...] = jnp.zeros_like(den)

        if emit_bf16:
            lhs = inc_ref[...]
            row = jax.lax.broadcasted_iota(jnp.int32, (bk, bm), 0)
            lhs = jnp.where(k * bk + row < N_NODES, lhs, 0.0)
            lhs = lhs.astype(jnp.bfloat16)
            cpy_ref[...] = lhs
        else:
            lhs = inc_ref[...]
        mxv = jnp.max(mx_ref[...], axis=0, keepdims=True)
        w = jnp.exp(l_ref[pl.ds(k * bk, bk), :] - mxv)
        wv = (w * v_ref[pl.ds(k * bk, bk), :]).astype(jnp.bfloat16)
        w = w.astype(jnp.bfloat16)
        num[...] += jax.lax.dot_general(lhs, wv, dn,
                                        preferred_element_type=jnp.float32)
        den[...] += jax.lax.dot_general(lhs, w, dn,
                                        preferred_element_type=jnp.float32)

        @pl.when(k == kt - 1)
        def _():
            s = s_ref[...]
            pooled = num[...] / (den[...] + 1e-9)
            y = _ln_rows(pooled, s[0:1, :], s[1:2, :])
            y2 = jnp.dot(
                jax.nn.relu(jnp.dot(y, w1_ref[...],
                                    preferred_element_type=jnp.float32)),
                w2_ref[...], preferred_element_type=jnp.float32)
            o = _ln_rows(y + y2, s[2:3, :], s[3:4, :])
            if classify:
                hcl = jax.nn.relu(
                    jnp.dot(o, wc1_ref[...],
                            preferred_element_type=jnp.float32) + s[4:5, :])
                o = jnp.dot(hcl, wc2_ref[...],
                            preferred_element_type=jnp.float32) + s[5:6, :]
            out_ref[...] = o

    nsrc = incT.shape[0]
    in_specs = [
        pl.BlockSpec((bk, bm), lambda m, k: (k, m)),
        pl.BlockSpec((NP, D), lambda m, k: (0, 0)),
        pl.BlockSpec((NP, D), lambda m, k: (0, 0)),
        pl.BlockSpec((8, D), lambda m, k: (0, 0)),
        pl.BlockSpec((D, D), lambda m, k: (0, 0)),
        pl.BlockSpec((D, D), lambda m, k: (0, 0)),
        pl.BlockSpec((8, D), lambda m, k: (0, 0)),
    ]
    args = [incT, l128, hv, mx, W1_b, W2_b, small]
    if classify:
        in_specs += [pl.BlockSpec((D, D), lambda m, k: (0, 0)),
                     pl.BlockSpec((D, D), lambda m, k: (0, 0))]
        args += [Wc1, Wc2p]

    out_specs = [pl.BlockSpec((bm, D), lambda m, k: (m, 0))]
    out_shape = [jax.ShapeDtypeStruct((N_EDGES, D), jnp.float32)]
    if emit_bf16:
        out_specs.append(pl.BlockSpec((bk, bm), lambda m, k: (k, m)))
        out_shape.append(jax.ShapeDtypeStruct((NP, N_EDGES), jnp.bfloat16))

    res = pl.pallas_call(
        body,
        grid=(mt, kt),
        in_specs=in_specs,
        out_specs=out_specs,
        out_shape=out_shape,
        scratch_shapes=[pltpu.VMEM((bm, D), jnp.float32),
                        pltpu.VMEM((bm, D), jnp.float32)],
    )(*args)
    return res if emit_bf16 else res[0]


def _pool_n_from_e(incT_bf16, l128, hv, mx, W1_b, W2_b, small, bm, bk):
    mt = NP // bm
    kt = N_EDGES // bk

    def body(inc_ref, l_ref, v_ref, mx_ref, w1_ref, w2_ref, s_ref,
             out_ref, num, den):
        k = pl.program_id(1)

        @pl.when(k == 0)
        def _():
            num[...] = jnp.zeros_like(num)
            den[...] = jnp.zeros_like(den)

        lhs = inc_ref[...]
        mxv = jnp.max(mx_ref[...], axis=0, keepdims=True)
        w = jnp.exp(l_ref[pl.ds(k * bk, bk), :] - mxv)
        wv = (w * v_ref[pl.ds(k * bk, bk), :]).astype(jnp.bfloat16)
        w = w.astype(jnp.bfloat16)
        num[...] += jnp.dot(lhs, wv, preferred_element_type=jnp.float32)
        den[...] += jnp.dot(lhs, w, preferred_element_type=jnp.float32)

        @pl.when(k == kt - 1)
        def _():
            s = s_ref[...]
            pooled = num[...] / (den[...] + 1e-9)
            y = _ln_rows(pooled, s[0:1, :], s[1:2, :])
            y2 = jnp.dot(
                jax.nn.relu(jnp.dot(y, w1_ref[...],
                                    preferred_element_type=jnp.float32)),
                w2_ref[...], preferred_element_type=jnp.float32)
            out_ref[...] = _ln_rows(y + y2, s[2:3, :], s[3:4, :])

    return pl.pallas_call(
        body,
        grid=(mt, kt),
        in_specs=[
            pl.BlockSpec((bm, bk), lambda m, k: (m, k)),
            pl.BlockSpec((N_EDGES, D), lambda m, k: (0, 0)),
            pl.BlockSpec((N_EDGES, D), lambda m, k: (0, 0)),
            pl.BlockSpec((8, D), lambda m, k: (0, 0)),
            pl.BlockSpec((D, D), lambda m, k: (0, 0)),
            pl.BlockSpec((D, D), lambda m, k: (0, 0)),
            pl.BlockSpec((8, D), lambda m, k: (0, 0)),
        ],
        out_specs=pl.BlockSpec((bm, D), lambda m, k: (m, 0)),
        out_shape=jax.ShapeDtypeStruct((NP, D), jnp.float32),
        scratch_shapes=[pltpu.VMEM((bm, D), jnp.float32),
                        pltpu.VMEM((bm, D), jnp.float32)],
    )(incT_bf16, l128, hv, mx, W1_b, W2_b, small)


def kernel(x, incidence_matrix, Wk, Wv, q, W1, W2, ln_g, ln_b,
           Wc1, bc1, Wc2, bc2):
    f32 = jnp.float32
    scale = jnp.sqrt(jnp.asarray(DH, f32))
    x_p = jnp.pad(x, ((0, NP - N_NODES), (0, 0)))
    incT = incidence_matrix.T

    def p128(b):
        qexp = jnp.zeros((H, DH, H), f32)
        qexp = qexp.at[jnp.arange(H), :, jnp.arange(H)].set(q[b])
        qexp = qexp.reshape(D, H)
        P = (Wk[b] @ qexp) / scale
        return jnp.repeat(P, DH, axis=1)

    zero = jnp.zeros((D,), f32)

    def small(b, classify=False):
        rows = [ln_g[b, 0], ln_b[b, 0], ln_g[b, 1], ln_b[b, 1]]
        if classify:
            rows += [bc1, jnp.broadcast_to(bc2, (D,))]
        else:
            rows += [zero, zero]
        rows += [zero, zero]
        return jnp.stack(rows)

    l0, v0, m0 = _src_prep(x_p, Wv[0], p128(0), N_NODES, 2048)
    h1, incT_bf16 = _pool_e_from_n(incT, l0, v0, m0, W1[0], W2[0], small(0),
                                   bm=1024, bk=2048, emit_bf16=True)
    l1, v1, m1 = _src_prep(h1, Wv[1], p128(1), N_EDGES, 2048)
    h0 = _pool_n_from_e(incT_bf16, l1, v1, m1, W1[1], W2[1],
                        small(1), bm=2048, bk=1024)
    l2, v2, m2 = _src_prep(h0, Wv[2], p128(2), N_NODES, 2048)
    Wc2p = jnp.pad(Wc2, ((0, 0), (0, D - 1)))
    res = _pool_e_from_n(incT_bf16, l2, v2, m2, W1[2], W2[2],
                         small(2, classify=True), bm=1024, bk=2048,
                         classify=True, Wc1=Wc1, Wc2p=Wc2p)
    return res[:, :1]


# speedup vs baseline: 1.9360x; 1.2391x over previous
import jax
import jax.numpy as jnp
from jax.experimental import pallas as pl
from jax.experimental.pallas import tpu as pltpu

N_NODES = 10000
N_EDGES = 4096
D = 128
H = 8
DH = 16
NP = 10240


def _ln_rows(xv, g, b):
    m = jnp.mean(xv, axis=-1, keepdims=True)
    v = jnp.mean((xv - m) ** 2, axis=-1, keepdims=True)
    return (xv - m) * jax.lax.rsqrt(v + 1e-5) * g + b


def _src_prep(h, Wv_b, P128, ns, bt):
    npad = h.shape[0]
    grid = (npad // bt,)

    def body(h_ref, wv_ref, p_ref, l_ref, v_ref, mx_ref):
        i = pl.program_id(0)
        h_ = h_ref[...]
        hv = jnp.dot(h_, wv_ref[...], preferred_element_type=jnp.float32)
        l = jnp.dot(h_, p_ref[...], preferred_element_type=jnp.float32)
        row = jax.lax.broadcasted_iota(jnp.int32, (bt, D), 0)
        valid = (i * bt + row) < ns
        l = jnp.where(valid, l, 0.0)
        hv = jnp.where(valid, hv, 0.0)
        l_ref[...] = l
        v_ref[...] = hv
        tmax = jnp.max(jnp.where(valid, l, -1e30), axis=0, keepdims=True)
        tmax = jnp.broadcast_to(tmax, (8, D))

        @pl.when(i == 0)
        def _():
            mx_ref[...] = jnp.full((8, D), -1e30, jnp.float32)

        mx_ref[...] = jnp.maximum(mx_ref[...], tmax)

    return pl.pallas_call(
        body,
        grid=grid,
        in_specs=[
            pl.BlockSpec((bt, D), lambda i: (i, 0)),
            pl.BlockSpec((D, D), lambda i: (0, 0)),
            pl.BlockSpec((D, D), lambda i: (0, 0)),
        ],
        out_specs=[
            pl.BlockSpec((bt, D), lambda i: (i, 0)),
            pl.BlockSpec((bt, D), lambda i: (i, 0)),
            pl.BlockSpec((8, D), lambda i: (0, 0)),
        ],
        out_shape=[
            jax.ShapeDtypeStruct((npad, D), jnp.float32),
            jax.ShapeDtypeStruct((npad, D), jnp.float32),
            jax.ShapeDtypeStruct((8, D), jnp.float32),
        ],
    )(h, Wv_b, P128)


def _pool_e_from_n(incT, l128, hv, mx, W1_b, W2_b, small, bm, bk,
                   emit_bf16=False, classify=False, Wc1=None, Wc2p=None):
    mt = N_EDGES // bm
    kt = NP // bk
    dn = (((0,), (0,)), ((), ()))

    def body(inc_ref, l_ref, v_ref, mx_ref, w1_ref, w2_ref, s_ref, *rest):
        if classify:
            wc1_ref, wc2_ref = rest[:2]
            rest = rest[2:]
        if emit_bf16:
            out_ref, cpy_ref, num, den = rest
        else:
            out_ref, num, den = rest
        k = pl.program_id(1)

        @pl.when(k == 0)
        def _():
            num[...] = jnp.zeros_like(num)
            den[...] = jnp.zeros_like(den)

        if emit_bf16:
            lhs = inc_ref[...]
            row = jax.lax.broadcasted_iota(jnp.int32, (bk, bm), 0)
            lhs = jnp.where(k * bk + row < N_NODES, lhs, 0.0)
            cpy_ref[...] = lhs.astype(jnp.float8_e4m3fn)
            lhs = lhs.astype(jnp.bfloat16)
        else:
            lhs = inc_ref[...]
        mxv = jnp.max(mx_ref[...], axis=0, keepdims=True)
        w = jnp.exp(l_ref[pl.ds(k * bk, bk), :] - mxv)
        rdt = lhs.dtype
        wv = (w * v_ref[pl.ds(k * bk, bk), :]).astype(rdt)
        w = w.astype(rdt)
        num[...] += jax.lax.dot_general(lhs, wv, dn,
                                        preferred_element_type=jnp.float32)
        den[...] += jax.lax.dot_general(lhs, w, dn,
                                        preferred_element_type=jnp.float32)

        @pl.when(k == kt - 1)
        def _():
            s = s_ref[...]
            pooled = num[...] / (den[...] + 1e-9)
            y = _ln_rows(pooled, s[0:1, :], s[1:2, :])
            y2 = jnp.dot(
                jax.nn.relu(jnp.dot(y, w1_ref[...],
                                    preferred_element_type=jnp.float32)),
                w2_ref[...], preferred_element_type=jnp.float32)
            o = _ln_rows(y + y2, s[2:3, :], s[3:4, :])
            if classify:
                hcl = jax.nn.relu(
                    jnp.dot(o, wc1_ref[...],
                            preferred_element_type=jnp.float32) + s[4:5, :])
                o = jnp.dot(hcl, wc2_ref[...],
                            preferred_element_type=jnp.float32) + s[5:6, :]
            out_ref[...] = o

    nsrc = incT.shape[0]
    in_specs = [
        pl.BlockSpec((bk, bm), lambda m, k: (k, m)),
        pl.BlockSpec((NP, D), lambda m, k: (0, 0)),
        pl.BlockSpec((NP, D), lambda m, k: (0, 0)),
        pl.BlockSpec((8, D), lambda m, k: (0, 0)),
        pl.BlockSpec((D, D), lambda m, k: (0, 0)),
        pl.BlockSpec((D, D), lambda m, k: (0, 0)),
        pl.BlockSpec((8, D), lambda m, k: (0, 0)),
    ]
    args = [incT, l128, hv, mx, W1_b, W2_b, small]
    if classify:
        in_specs += [pl.BlockSpec((D, D), lambda m, k: (0, 0)),
                     pl.BlockSpec((D, D), lambda m, k: (0, 0))]
        args += [Wc1, Wc2p]

    out_specs = [pl.BlockSpec((bm, D), lambda m, k: (m, 0))]
    out_shape = [jax.ShapeDtypeStruct((N_EDGES, D), jnp.float32)]
    if emit_bf16:
        out_specs.append(pl.BlockSpec((bk, bm), lambda m, k: (k, m)))
        out_shape.append(jax.ShapeDtypeStruct((NP, N_EDGES), jnp.float8_e4m3fn))

    res = pl.pallas_call(
        body,
        grid=(mt, kt),
        in_specs=in_specs,
        out_specs=out_specs,
        out_shape=out_shape,
        scratch_shapes=[pltpu.VMEM((bm, D), jnp.float32),
                        pltpu.VMEM((bm, D), jnp.float32)],
    )(*args)
    return res if emit_bf16 else res[0]


def _pool_n_from_e(incT_bf16, l128, hv, mx, W1_b, W2_b, small, bm, bk):
    mt = NP // bm
    kt = N_EDGES // bk

    def body(inc_ref, l_ref, v_ref, mx_ref, w1_ref, w2_ref, s_ref,
             out_ref, num, den):
        k = pl.program_id(1)

        @pl.when(k == 0)
        def _():
            num[...] = jnp.zeros_like(num)
            den[...] = jnp.zeros_like(den)

        lhs = inc_ref[...]
        mxv = jnp.max(mx_ref[...], axis=0, keepdims=True)
        w = jnp.exp(l_ref[pl.ds(k * bk, bk), :] - mxv)
        wv = (w * v_ref[pl.ds(k * bk, bk), :]).astype(lhs.dtype)
        w = w.astype(lhs.dtype)
        num[...] += jnp.dot(lhs, wv, preferred_element_type=jnp.float32)
        den[...] += jnp.dot(lhs, w, preferred_element_type=jnp.float32)

        @pl.when(k == kt - 1)
        def _():
            s = s_ref[...]
            pooled = num[...] / (den[...] + 1e-9)
            y = _ln_rows(pooled, s[0:1, :], s[1:2, :])
            y2 = jnp.dot(
                jax.nn.relu(jnp.dot(y, w1_ref[...],
                                    preferred_element_type=jnp.float32)),
                w2_ref[...], preferred_element_type=jnp.float32)
            out_ref[...] = _ln_rows(y + y2, s[2:3, :], s[3:4, :])

    return pl.pallas_call(
        body,
        grid=(mt, kt),
        in_specs=[
            pl.BlockSpec((bm, bk), lambda m, k: (m, k)),
            pl.BlockSpec((N_EDGES, D), lambda m, k: (0, 0)),
            pl.BlockSpec((N_EDGES, D), lambda m, k: (0, 0)),
            pl.BlockSpec((8, D), lambda m, k: (0, 0)),
            pl.BlockSpec((D, D), lambda m, k: (0, 0)),
            pl.BlockSpec((D, D), lambda m, k: (0, 0)),
            pl.BlockSpec((8, D), lambda m, k: (0, 0)),
        ],
        out_specs=pl.BlockSpec((bm, D), lambda m, k: (m, 0)),
        out_shape=jax.ShapeDtypeStruct((NP, D), jnp.float32),
        scratch_shapes=[pltpu.VMEM((bm, D), jnp.float32),
                        pltpu.VMEM((bm, D), jnp.float32)],
    )(incT_bf16, l128, hv, mx, W1_b, W2_b, small)


def kernel(x, incidence_matrix, Wk, Wv, q, W1, W2, ln_g, ln_b,
           Wc1, bc1, Wc2, bc2):
    f32 = jnp.float32
    scale = jnp.sqrt(jnp.asarray(DH, f32))
    x_p = jnp.pad(x, ((0, NP - N_NODES), (0, 0)))
    incT = incidence_matrix.T

    def p128(b):
        qexp = jnp.zeros((H, DH, H), f32)
        qexp = qexp.at[jnp.arange(H), :, jnp.arange(H)].set(q[b])
        qexp = qexp.reshape(D, H)
        P = (Wk[b] @ qexp) / scale
        return jnp.repeat(P, DH, axis=1)

    zero = jnp.zeros((D,), f32)

    def small(b, classify=False):
        rows = [ln_g[b, 0], ln_b[b, 0], ln_g[b, 1], ln_b[b, 1]]
        if classify:
            rows += [bc1, jnp.broadcast_to(bc2, (D,))]
        else:
            rows += [zero, zero]
        rows += [zero, zero]
        return jnp.stack(rows)

    l0, v0, m0 = _src_prep(x_p, Wv[0], p128(0), N_NODES, 2048)
    h1, incT_bf16 = _pool_e_from_n(incT, l0, v0, m0, W1[0], W2[0], small(0),
                                   bm=1024, bk=2048, emit_bf16=True)
    l1, v1, m1 = _src_prep(h1, Wv[1], p128(1), N_EDGES, 2048)
    h0 = _pool_n_from_e(incT_bf16, l1, v1, m1, W1[1], W2[1],
                        small(1), bm=2048, bk=1024)
    l2, v2, m2 = _src_prep(h0, Wv[2], p128(2), N_NODES, 2048)
    Wc2p = jnp.pad(Wc2, ((0, 0), (0, D - 1)))
    res = _pool_e_from_n(incT_bf16, l2, v2, m2, W1[2], W2[2],
                         small(2, classify=True), bm=1024, bk=2048,
                         classify=True, Wc1=Wc1, Wc2p=Wc2p)
    return res[:, :1]
